# Initial kernel scaffold; baseline (speedup 1.0000x reference)
import jax
import jax.numpy as jnp
from jax import lax
from jax.experimental import pallas as pl
from jax.experimental.pallas import tpu as pltpu
from jax.experimental.pallas import tpu_sc as plsc

N = 10000
E = 320000
D = 128
CHUNK = 128
NCHUNKS = E // CHUNK
NTILES = 32
CPT = NCHUNKS // NTILES
EXTRA = NCHUNKS - CPT * NTILES
ROWS_PT = N // 16
DEGW = 16


def _sc_agg(with_deg):
    out_type = [jax.ShapeDtypeStruct((2, N, D), jnp.float32)]
    scratch = [
        pltpu.VMEM((CPT + 1, CHUNK), jnp.int32),
        pltpu.VMEM((CPT + 1, CHUNK), jnp.int32),
        pltpu.VMEM((CHUNK, D), jnp.float32),
        pltpu.VMEM_SHARED((N, D), jnp.float32),
    ]
    if with_deg:
        out_type.append(jax.ShapeDtypeStruct((2, N, DEGW), jnp.float32))
        scratch.append(pltpu.VMEM((CHUNK, DEGW), jnp.float32))
        scratch.append(pltpu.VMEM_SHARED((N, DEGW), jnp.float32))

    mesh = plsc.VectorSubcoreMesh(core_axis_name="c", subcore_axis_name="s")

    def body(h_hbm, src_hbm, dst_hbm, *rest):
        if with_deg:
            agg_out, deg_out, src_v, dst_v, rows_v, agg_sh, ones_v, deg_sh = rest
        else:
            (agg_out, src_v, dst_v, rows_v, agg_sh) = rest
        c = lax.axis_index("c")
        s = lax.axis_index("s")
        w = c * 16 + s

        @pl.loop(0, CHUNK)
        def _(i):
            @pl.loop(0, D, step=16)
            def _(j):
                rows_v[i, pl.ds(j, 16)] = jnp.zeros((16,), jnp.float32)
            if with_deg:
                ones_v[i, :] = jnp.zeros((DEGW,), jnp.float32)

        base = s * ROWS_PT
        for k in range(4):
            pltpu.sync_copy(rows_v, agg_sh.at[pl.ds(base + k * CHUNK, CHUNK)])
            if with_deg:
                pltpu.sync_copy(ones_v,
                                deg_sh.at[pl.ds(base + k * CHUNK, CHUNK)])
        tail = ROWS_PT - 4 * CHUNK
        pltpu.sync_copy(rows_v.at[pl.ds(0, tail)],
                        agg_sh.at[pl.ds(base + 4 * CHUNK, tail)])
        if with_deg:
            pltpu.sync_copy(ones_v.at[pl.ds(0, tail)],
                            deg_sh.at[pl.ds(base + 4 * CHUNK, tail)])

            @pl.loop(0, CHUNK)
            def _(i):
                ones_v[i, :] = jnp.ones((DEGW,), jnp.float32)

        cbase = w * CPT
        pltpu.sync_copy(src_hbm.at[pl.ds(cbase, CPT)], src_v.at[pl.ds(0, CPT)])
        pltpu.sync_copy(dst_hbm.at[pl.ds(cbase, CPT)], dst_v.at[pl.ds(0, CPT)])

        @pl.when(w < EXTRA)
        def _():
            xc = NTILES * CPT + w
            pltpu.sync_copy(src_hbm.at[pl.ds(xc, 1)], src_v.at[pl.ds(CPT, 1)])
            pltpu.sync_copy(dst_hbm.at[pl.ds(xc, 1)], dst_v.at[pl.ds(CPT, 1)])

        plsc.subcore_barrier()

        def do_chunk(j):
            pltpu.sync_copy(h_hbm.at[src_v.at[j]], rows_v)
            pltpu.sync_copy(rows_v, agg_sh.at[dst_v.at[j]], add=True)
            if with_deg:
                pltpu.sync_copy(ones_v, deg_sh.at[dst_v.at[j]], add=True)

        @pl.loop(0, CPT)
        def _(j):
            do_chunk(j)

        @pl.when(w < EXTRA)
        def _():
            do_chunk(CPT)

        plsc.subcore_barrier()

        pltpu.sync_copy(agg_sh.at[pl.ds(base, ROWS_PT)],
                        agg_out.at[c].at[pl.ds(base, ROWS_PT)])
        if with_deg:
            pltpu.sync_copy(deg_sh.at[pl.ds(base, ROWS_PT)],
                            deg_out.at[c].at[pl.ds(base, ROWS_PT)])

    return pl.kernel(body, out_type=out_type, mesh=mesh,
                     scratch_types=scratch)


_sc_agg_deg_kernel = _sc_agg(with_deg=True)
_sc_agg_kernel = _sc_agg(with_deg=False)


def _tc_body(h_ref, a0_ref, a1_ref, d_ref, ws_ref, wn_ref, b_ref, o_ref):
    agg = a0_ref[...] + a1_ref[...]
    deg = d_ref[0, :, 0:1] + d_ref[1, :, 0:1]
    mean = agg / jnp.maximum(deg, 1.0)
    out = jnp.dot(h_ref[...], ws_ref[...], preferred_element_type=jnp.float32)
    out = out + jnp.dot(mean, wn_ref[...],
                        preferred_element_type=jnp.float32)
    out = out + b_ref[...]
    o_ref[...] = jnp.maximum(out, 0.0)


def _tc_combine(h, agg, degp, W_self, W_neigh, b):
    R = 1000
    return pl.pallas_call(
        _tc_body,
        grid=(N // R,),
        in_specs=[
            pl.BlockSpec((R, D), lambda i: (i, 0)),
            pl.BlockSpec((R, D), lambda i: (i, 0)),
            pl.BlockSpec((R, D), lambda i: (i, 0)),
            pl.BlockSpec((2, R, DEGW), lambda i: (0, i, 0)),
            pl.BlockSpec((D, D), lambda i: (0, 0)),
            pl.BlockSpec((D, D), lambda i: (0, 0)),
            pl.BlockSpec((1, D), lambda i: (0, 0)),
        ],
        out_specs=pl.BlockSpec((R, D), lambda i: (i, 0)),
        out_shape=jax.ShapeDtypeStruct((N, D), jnp.float32),
    )(h, agg[0], agg[1], degp, W_self, W_neigh, b.reshape(1, D))


def kernel(x, edge_index, W1_self, W1_neigh, b1, W2_self, W2_neigh, b2):
    src = edge_index[0].astype(jnp.int32).reshape(NCHUNKS, CHUNK)
    dst = edge_index[1].astype(jnp.int32).reshape(NCHUNKS, CHUNK)
    agg1, degp = _sc_agg_deg_kernel(x, src, dst)
    h1 = _tc_combine(x, agg1, degp, W1_self, W1_neigh, b1)
    (agg2,) = _sc_agg_kernel(h1, src, dst)
    h2 = _tc_combine(h1, agg2, degp, W2_self, W2_neigh, b2)
    return h2


# speedup vs baseline: 7.5824x; 7.5824x over previous
import jax
import jax.numpy as jnp
from jax import lax
from jax.experimental import pallas as pl
from jax.experimental.pallas import tpu as pltpu
from jax.experimental.pallas import tpu_sc as plsc

N = 10000
E = 320000
D = 128
CHUNK = 128
NTILES = 32
CPT = 80
NCHUNKS = NTILES * CPT
EP = NCHUNKS * CHUNK
SH_ROWS = 10112
ZPT = SH_ROWS // 16
WPT = 624
DEGW = 128

_MESH = plsc.VectorSubcoreMesh(core_axis_name="c", subcore_axis_name="s")


def _zero_vmem_rows(ref, nrows, width):
    @pl.loop(0, nrows)
    def _(i):
        @pl.loop(0, width, step=16)
        def _(j):
            ref[i, pl.ds(j, 16)] = jnp.zeros((16,), jnp.float32)


def _zero_shared(zsrc, shared, zbase):
    for k in range(4):
        pltpu.sync_copy(zsrc, shared.at[pl.ds(zbase + k * CHUNK, CHUNK)])
    tail = ZPT - 4 * CHUNK
    pltpu.sync_copy(zsrc.at[pl.ds(0, tail)],
                    shared.at[pl.ds(zbase + 4 * CHUNK, tail)])


def _writeout(shared, out, c, s):
    wbase = s * WPT
    pltpu.sync_copy(shared.at[pl.ds(wbase, WPT)],
                    out.at[c].at[pl.ds(wbase, WPT)])

    @pl.when(s == 15)
    def _():
        last = 16 * WPT
        pltpu.sync_copy(shared.at[pl.ds(last, N - last)],
                        out.at[c].at[pl.ds(last, N - last)])


def _sc_deg_body(dst_hbm, deg_out, dst_v, ones_v, deg_sh):
    c = lax.axis_index("c")
    s = lax.axis_index("s")
    w = c * 16 + s

    _zero_vmem_rows(ones_v, CHUNK, DEGW)
    _zero_shared(ones_v, deg_sh, s * ZPT)

    @pl.loop(0, CHUNK)
    def _(i):
        ones_v[i, :] = jnp.ones((DEGW,), jnp.float32)

    pltpu.sync_copy(dst_hbm.at[pl.ds(w * CPT, CPT)], dst_v)
    plsc.subcore_barrier()

    @pl.loop(0, CPT)
    def _(j):
        pltpu.sync_copy(ones_v, deg_sh.at[dst_v.at[j]], add=True)

    plsc.subcore_barrier()
    _writeout(deg_sh, deg_out, c, s)


_sc_deg_kernel = pl.kernel(
    _sc_deg_body,
    out_type=[jax.ShapeDtypeStruct((2, N, DEGW), jnp.float32)],
    mesh=_MESH,
    scratch_types=[
        pltpu.VMEM((CPT, CHUNK), jnp.int32),
        pltpu.VMEM((CHUNK, DEGW), jnp.float32),
        pltpu.VMEM_SHARED((SH_ROWS, DEGW), jnp.float32),
    ],
)


def _sc_agg_body(h_hbm, src_hbm, dst_hbm, agg_out, src_v, dst_v, rows_v,
                 agg_sh):
    c = lax.axis_index("c")
    s = lax.axis_index("s")
    w = c * 16 + s

    _zero_vmem_rows(rows_v, CHUNK, D)
    _zero_shared(rows_v, agg_sh, s * ZPT)

    pltpu.sync_copy(src_hbm.at[pl.ds(w * CPT, CPT)], src_v)
    pltpu.sync_copy(dst_hbm.at[pl.ds(w * CPT, CPT)], dst_v)
    plsc.subcore_barrier()

    @pl.loop(0, CPT)
    def _(j):
        pltpu.sync_copy(h_hbm.at[src_v.at[j]], rows_v)
        pltpu.sync_copy(rows_v, agg_sh.at[dst_v.at[j]], add=True)

    plsc.subcore_barrier()
    _writeout(agg_sh, agg_out, c, s)


_sc_agg_kernel = pl.kernel(
    _sc_agg_body,
    out_type=[jax.ShapeDtypeStruct((2, N, D), jnp.float32)],
    mesh=_MESH,
    scratch_types=[
        pltpu.VMEM((CPT, CHUNK), jnp.int32),
        pltpu.VMEM((CPT, CHUNK), jnp.int32),
        pltpu.VMEM((CHUNK, D), jnp.float32),
        pltpu.VMEM_SHARED((SH_ROWS, D), jnp.float32),
    ],
)


def _tc_body(h_ref, a0_ref, a1_ref, d_ref, ws_ref, wn_ref, b_ref, o_ref):
    agg = a0_ref[...] + a1_ref[...]
    deg = d_ref[0, :, 0:1] + d_ref[1, :, 0:1]
    mean = agg / jnp.maximum(deg, 1.0)
    out = jnp.dot(h_ref[...], ws_ref[...], preferred_element_type=jnp.float32,
                  precision=jax.lax.Precision.HIGHEST)
    out = out + jnp.dot(mean, wn_ref[...],
                        preferred_element_type=jnp.float32,
                        precision=jax.lax.Precision.HIGHEST)
    out = out + b_ref[...]
    o_ref[...] = jnp.maximum(out, 0.0)


def _tc_combine(h, agg, degp, W_self, W_neigh, b):
    R = 1000
    return pl.pallas_call(
        _tc_body,
        grid=(N // R,),
        in_specs=[
            pl.BlockSpec((R, D), lambda i: (i, 0)),
            pl.BlockSpec((R, D), lambda i: (i, 0)),
            pl.BlockSpec((R, D), lambda i: (i, 0)),
            pl.BlockSpec((2, R, DEGW), lambda i: (0, i, 0)),
            pl.BlockSpec((D, D), lambda i: (0, 0)),
            pl.BlockSpec((D, D), lambda i: (0, 0)),
            pl.BlockSpec((1, D), lambda i: (0, 0)),
        ],
        out_specs=pl.BlockSpec((R, D), lambda i: (i, 0)),
        out_shape=jax.ShapeDtypeStruct((N, D), jnp.float32),
    )(h, agg[0], agg[1], degp, W_self, W_neigh, b.reshape(1, D))


def _pad_edges(edge_index):
    npad = EP - E
    pad_src = jnp.arange(npad, dtype=jnp.int32) % N
    pad_dst = N + (jnp.arange(npad, dtype=jnp.int32) % 16)
    src = jnp.concatenate([edge_index[0].astype(jnp.int32), pad_src])
    dst = jnp.concatenate([edge_index[1].astype(jnp.int32), pad_dst])
    return src.reshape(NCHUNKS, CHUNK), dst.reshape(NCHUNKS, CHUNK)


def kernel(x, edge_index, W1_self, W1_neigh, b1, W2_self, W2_neigh, b2):
    src, dst = _pad_edges(edge_index)
    (degp,) = _sc_deg_kernel(dst)
    (agg1,) = _sc_agg_kernel(x, src, dst)
    h1 = _tc_combine(x, agg1, degp, W1_self, W1_neigh, b1)
    (agg2,) = _sc_agg_kernel(h1, src, dst)
    h2 = _tc_combine(h1, agg2, degp, W2_self, W2_neigh, b2)
    return h2


# speedup vs baseline: 10.1428x; 1.3377x over previous
import jax
import jax.numpy as jnp
from jax import lax
from jax.experimental import pallas as pl
from jax.experimental.pallas import tpu as pltpu
from jax.experimental.pallas import tpu_sc as plsc

N = 10000
E = 320000
D = 128
CHUNK = 128
NTILES = 32
CPT = 80
NCHUNKS = NTILES * CPT
EP = NCHUNKS * CHUNK
SH_ROWS = 10112
ZPT = SH_ROWS // 16
WPT = 624
DEGW = 128

_MESH = plsc.VectorSubcoreMesh(core_axis_name="c", subcore_axis_name="s")


def _zero_vmem_rows(ref, nrows, width):
    @pl.loop(0, nrows)
    def _(i):
        @pl.loop(0, width, step=16)
        def _(j):
            ref[i, pl.ds(j, 16)] = jnp.zeros((16,), jnp.float32)


def _zero_shared(zsrc, shared, zbase):
    for k in range(4):
        pltpu.sync_copy(zsrc, shared.at[pl.ds(zbase + k * CHUNK, CHUNK)])
    tail = ZPT - 4 * CHUNK
    pltpu.sync_copy(zsrc.at[pl.ds(0, tail)],
                    shared.at[pl.ds(zbase + 4 * CHUNK, tail)])


def _writeout(shared, out, c, s):
    wbase = s * WPT
    pltpu.sync_copy(shared.at[pl.ds(wbase, WPT)],
                    out.at[c].at[pl.ds(wbase, WPT)])

    @pl.when(s == 15)
    def _():
        last = 16 * WPT
        pltpu.sync_copy(shared.at[pl.ds(last, N - last)],
                        out.at[c].at[pl.ds(last, N - last)])


def _sc_deg_body(dst_hbm, deg_out, dst_v, ones_v, sem, deg_sh):
    c = lax.axis_index("c")
    s = lax.axis_index("s")
    w = c * 16 + s

    _zero_vmem_rows(ones_v, CHUNK, DEGW)
    _zero_shared(ones_v, deg_sh, s * ZPT)

    @pl.loop(0, CHUNK)
    def _(i):
        ones_v[i, :] = jnp.ones((DEGW,), jnp.float32)

    pltpu.sync_copy(dst_hbm.at[pl.ds(w * CPT, CPT)], dst_v)
    plsc.subcore_barrier()

    @pl.loop(0, CPT)
    def _(j):
        pltpu.async_copy(ones_v, deg_sh.at[dst_v.at[j]], sem, add=True)

    @pl.loop(0, CPT)
    def _(j):
        pltpu.make_async_copy(ones_v, deg_sh.at[dst_v.at[j]], sem).wait()

    plsc.subcore_barrier()
    _writeout(deg_sh, deg_out, c, s)


_sc_deg_kernel = pl.kernel(
    _sc_deg_body,
    out_type=[jax.ShapeDtypeStruct((2, N, DEGW), jnp.float32)],
    mesh=_MESH,
    scratch_types=[
        pltpu.VMEM((CPT, CHUNK), jnp.int32),
        pltpu.VMEM((CHUNK, DEGW), jnp.float32),
        pltpu.SemaphoreType.DMA,
        pltpu.VMEM_SHARED((SH_ROWS, DEGW), jnp.float32),
    ],
)

HPASS = CPT // 2


def _sc_agg_body(h_hbm, src_hbm, dst_hbm, agg_out, src_v, dst_v, rows_a,
                 rows_b, sem_a, sem_b, agg_sh):
    c = lax.axis_index("c")
    s = lax.axis_index("s")
    w = c * 16 + s

    _zero_vmem_rows(rows_a, CHUNK, D)
    _zero_shared(rows_a, agg_sh, s * ZPT)
    plsc.subcore_barrier()

    for half in range(2):
        base = w * CPT + half * HPASS
        pltpu.sync_copy(src_hbm.at[pl.ds(base, HPASS)], src_v)
        pltpu.sync_copy(dst_hbm.at[pl.ds(base, HPASS)], dst_v)

        pltpu.async_copy(h_hbm.at[src_v.at[0]], rows_a, sem_a)

        @pl.loop(0, HPASS, step=2)
        def _(j):
            pltpu.make_async_copy(h_hbm.at[src_v.at[j]], rows_a,
                                  sem_a).wait()
            pltpu.async_copy(h_hbm.at[src_v.at[j + 1]], rows_b, sem_b)
            pltpu.sync_copy(rows_a, agg_sh.at[dst_v.at[j]], add=True)

            @pl.when(j + 2 < HPASS)
            def _():
                pltpu.async_copy(h_hbm.at[src_v.at[j + 2]], rows_a, sem_a)

            pltpu.make_async_copy(h_hbm.at[src_v.at[j + 1]], rows_b,
                                  sem_b).wait()
            pltpu.sync_copy(rows_b, agg_sh.at[dst_v.at[j + 1]], add=True)

    plsc.subcore_barrier()
    _writeout(agg_sh, agg_out, c, s)


_sc_agg_kernel = pl.kernel(
    _sc_agg_body,
    out_type=[jax.ShapeDtypeStruct((2, N, D), jnp.float32)],
    mesh=_MESH,
    scratch_types=[
        pltpu.VMEM((HPASS, CHUNK), jnp.int32),
        pltpu.VMEM((HPASS, CHUNK), jnp.int32),
        pltpu.VMEM((CHUNK, D), jnp.float32),
        pltpu.VMEM((CHUNK, D), jnp.float32),
        pltpu.SemaphoreType.DMA,
        pltpu.SemaphoreType.DMA,
        pltpu.VMEM_SHARED((SH_ROWS, D), jnp.float32),
    ],
)


def _tc_body(h_ref, a0_ref, a1_ref, d_ref, ws_ref, wn_ref, b_ref, o_ref):
    agg = a0_ref[...] + a1_ref[...]
    deg = d_ref[0, :, 0:1] + d_ref[1, :, 0:1]
    mean = agg / jnp.maximum(deg, 1.0)
    out = jnp.dot(h_ref[...], ws_ref[...], preferred_element_type=jnp.float32,
                  precision=jax.lax.Precision.HIGHEST)
    out = out + jnp.dot(mean, wn_ref[...],
                        preferred_element_type=jnp.float32,
                        precision=jax.lax.Precision.HIGHEST)
    out = out + b_ref[...]
    o_ref[...] = jnp.maximum(out, 0.0)


def _tc_combine(h, agg, degp, W_self, W_neigh, b):
    R = 1000
    return pl.pallas_call(
        _tc_body,
        grid=(N // R,),
        in_specs=[
            pl.BlockSpec((R, D), lambda i: (i, 0)),
            pl.BlockSpec((R, D), lambda i: (i, 0)),
            pl.BlockSpec((R, D), lambda i: (i, 0)),
            pl.BlockSpec((2, R, DEGW), lambda i: (0, i, 0)),
            pl.BlockSpec((D, D), lambda i: (0, 0)),
            pl.BlockSpec((D, D), lambda i: (0, 0)),
            pl.BlockSpec((1, D), lambda i: (0, 0)),
        ],
        out_specs=pl.BlockSpec((R, D), lambda i: (i, 0)),
        out_shape=jax.ShapeDtypeStruct((N, D), jnp.float32),
    )(h, agg[0], agg[1], degp, W_self, W_neigh, b.reshape(1, D))


def _pad_edges(edge_index):
    npad = EP - E
    pad_src = jnp.arange(npad, dtype=jnp.int32) % N
    pad_dst = N + (jnp.arange(npad, dtype=jnp.int32) % 16)
    src = jnp.concatenate([edge_index[0].astype(jnp.int32), pad_src])
    dst = jnp.concatenate([edge_index[1].astype(jnp.int32), pad_dst])
    return src.reshape(NCHUNKS, CHUNK), dst.reshape(NCHUNKS, CHUNK)


def kernel(x, edge_index, W1_self, W1_neigh, b1, W2_self, W2_neigh, b2):
    src, dst = _pad_edges(edge_index)
    (degp,) = _sc_deg_kernel(dst)
    (agg1,) = _sc_agg_kernel(x, src, dst)
    h1 = _tc_combine(x, agg1, degp, W1_self, W1_neigh, b1)
    (agg2,) = _sc_agg_kernel(h1, src, dst)
    h2 = _tc_combine(h1, agg2, degp, W2_self, W2_neigh, b2)
    return h2
